# Initial kernel scaffold; baseline (speedup 1.0000x reference)
import jax
import jax.numpy as jnp
from jax.experimental import pallas as pl


def kernel(x, edge_index, W, W_phi, bias):
    raise NotImplementedError("write your pallas kernel here")


# speedup vs baseline: 9.1653x; 9.1653x over previous
import functools

import jax
import jax.numpy as jnp
from jax import lax
from jax.experimental import pallas as pl
from jax.experimental.pallas import tpu as pltpu
from jax.experimental.pallas import tpu_sc as plsc

GAMMA = 0.1
EPSILON = 0.1

NC = 2
NS = 16
K = 128


@functools.cache
def _sc_mesh():
    return plsc.VectorSubcoreMesh(core_axis_name="core",
                                  subcore_axis_name="subcore",
                                  num_cores=NC, num_subcores=NS)


def _deg_body(npad, nch_deg, cols_hbm, ones_hbm, zeros_hbm, degp_hbm,
              cols_v, ones_v, zbuf, deg_sh):
    stripe = npad // NS
    c = lax.axis_index("core")
    s = lax.axis_index("subcore")
    pltpu.sync_copy(zeros_hbm, zbuf)
    pltpu.sync_copy(zbuf, deg_sh.at[pl.ds(s * stripe, stripe)])
    pltpu.sync_copy(cols_hbm.at[c, s], cols_v)
    pltpu.sync_copy(ones_hbm, ones_v)
    plsc.subcore_barrier()

    @pl.loop(0, nch_deg)
    def _(j):
        pltpu.sync_copy(ones_v, deg_sh.at[cols_v.at[j]], add=True)

    plsc.subcore_barrier()
    pltpu.sync_copy(deg_sh.at[pl.ds(s * stripe, stripe)], zbuf)
    pltpu.sync_copy(zbuf, degp_hbm.at[pl.ds(c * npad + s * stripe, stripe)])


def _gcn_body(npad, nch, xws_hbm, idx_hbm, zeros_hbm, gcn_hbm,
              i0, i1, g0, g1, acc_sh, is0, is1, gs0, gs1):
    stripe = npad // NS
    half = g0.shape[1]
    c = lax.axis_index("core")
    s = lax.axis_index("subcore")
    pltpu.sync_copy(zeros_hbm, g0)

    @pl.loop(0, stripe, step=K)
    def _(i):
        pltpu.sync_copy(g0, acc_sh.at[pl.ds(s * stripe + i, K)])

    plsc.subcore_barrier()

    pltpu.async_copy(idx_hbm.at[c, s, 0], i0, is0).wait()
    pltpu.async_copy(xws_hbm.at[i0.at[0]], g0, gs0)
    pltpu.async_copy(idx_hbm.at[c, s, 1], i1, is1)

    @pl.loop(0, nch, step=2)
    def _(j):
        pltpu.make_async_copy(idx_hbm.at[c, s, j + 1], i1, is1).wait()
        pltpu.make_async_copy(xws_hbm.at[i0.at[0]], g0, gs0).wait()
        pltpu.async_copy(xws_hbm.at[i1.at[0]], g1, gs1)
        pltpu.sync_copy(g0, acc_sh.at[i0.at[1]], add=True)

        @pl.when(j + 2 < nch)
        def _():
            pltpu.async_copy(idx_hbm.at[c, s, j + 2], i0, is0).wait()
            pltpu.async_copy(xws_hbm.at[i0.at[0]], g0, gs0)

        pltpu.make_async_copy(xws_hbm.at[i1.at[0]], g1, gs1).wait()
        pltpu.sync_copy(g1, acc_sh.at[i1.at[1]], add=True)

        @pl.when(j + 3 < nch)
        def _():
            pltpu.async_copy(idx_hbm.at[c, s, j + 3], i1, is1)

    plsc.subcore_barrier()

    @pl.loop(0, stripe, step=2 * K)
    def _(i):
        pltpu.sync_copy(acc_sh.at[pl.ds(s * stripe + i, K)], g0)
        pltpu.sync_copy(g0, gcn_hbm.at[c, pl.ds(s * stripe + i, K)])
        pltpu.sync_copy(acc_sh.at[pl.ds(s * stripe + i + K, K)], g1)
        pltpu.sync_copy(g1, gcn_hbm.at[c, pl.ds(s * stripe + i + K, K)])


def _dense_body(x_ref, wcat_ref, degp_ref, h2_ref, xws_ref):
    xb = x_ref[...]
    m = jnp.dot(xb, wcat_ref[...], preferred_element_type=jnp.float32)
    d = xb.shape[1]
    h2_ref[...] = m[:, d:]
    deg = degp_ref[:, 0:1] + degp_ref[:, 1:2]
    dis = jnp.where(deg > 0.0, lax.rsqrt(deg), 0.0)
    xw = m[:, :d] * dis
    half = d // 2
    xws_ref[0] = xw[:, :half]
    xws_ref[1] = xw[:, half:]


def _combine_body(x_ref, h2_ref, gcn_ref, degp_ref, bias_ref, o_ref):
    deg = degp_ref[:, 0:1] + degp_ref[:, 1:2]
    dis = jnp.where(deg > 0.0, lax.rsqrt(deg), 0.0)
    g = jnp.concatenate([gcn_ref[0], gcn_ref[1]], axis=1)
    h = h2_ref[...] + g * dis + bias_ref[...]
    o_ref[...] = x_ref[...] + EPSILON * jnp.tanh(h)


def kernel(x, edge_index, W, W_phi, bias):
    n, d = x.shape
    e = edge_index.shape[1]
    half = d // 2
    npad = ((n + K * NS - 1) // (K * NS)) * (K * NS)
    stripe = npad // NS
    nch = 2 * ((e + 2 * NS * K - 1) // (2 * NS * K))
    epad = NS * K * nch
    nch_deg = epad // (NC * NS * K)

    ei = edge_index.astype(jnp.int32)
    rows = jnp.concatenate([ei[0], jnp.zeros((epad - e,), jnp.int32)])
    cols = jnp.concatenate([ei[1],
                            jnp.full((epad - e,), npad - 1, jnp.int32)])
    rows3 = rows.reshape(NS, nch, K)
    cols3 = cols.reshape(NS, nch, K)
    idx_pack = jnp.stack(
        [jnp.stack([rows3, cols3], axis=2),
         jnp.stack([rows3 + n, cols3], axis=2)], axis=0)
    cols_deg = cols.reshape(NC, NS, nch_deg, K)

    ones128 = jnp.ones((K,), jnp.float32)
    zeros1 = jnp.zeros((stripe,), jnp.float32)
    zeros2 = jnp.zeros((K, half), jnp.float32)

    wcat = jnp.concatenate(
        [W_phi.T, (W - W.T - GAMMA * jnp.eye(d, dtype=x.dtype)).T], axis=1)

    deg_call = pl.kernel(
        functools.partial(_deg_body, npad, nch_deg),
        out_type=jax.ShapeDtypeStruct((NC * npad,), jnp.float32),
        mesh=_sc_mesh(),
        scratch_types=[
            pltpu.VMEM((nch_deg, K), jnp.int32),
            pltpu.VMEM((K,), jnp.float32),
            pltpu.VMEM((stripe,), jnp.float32),
            pltpu.VMEM_SHARED((npad,), jnp.float32),
        ],
    )
    degp = deg_call(cols_deg, ones128, zeros1)
    degp_t = degp.reshape(NC, npad).T

    nb = 10
    r = n // nb
    h2, xws = pl.pallas_call(
        _dense_body,
        grid=(nb,),
        in_specs=[
            pl.BlockSpec((r, d), lambda i: (i, 0)),
            pl.BlockSpec((d, 2 * d), lambda i: (0, 0)),
            pl.BlockSpec((r, 2), lambda i: (i, 0)),
        ],
        out_specs=[
            pl.BlockSpec((r, d), lambda i: (i, 0)),
            pl.BlockSpec((2, r, half), lambda i: (0, i, 0)),
        ],
        out_shape=[
            jax.ShapeDtypeStruct((n, d), jnp.float32),
            jax.ShapeDtypeStruct((2, n, half), jnp.float32),
        ],
    )(x, wcat, degp_t)

    gcn_call = pl.kernel(
        functools.partial(_gcn_body, npad, nch),
        out_type=jax.ShapeDtypeStruct((NC, npad, half), jnp.float32),
        mesh=_sc_mesh(),
        scratch_types=[
            pltpu.VMEM((2, K), jnp.int32),
            pltpu.VMEM((2, K), jnp.int32),
            pltpu.VMEM((K, half), jnp.float32),
            pltpu.VMEM((K, half), jnp.float32),
            pltpu.VMEM_SHARED((npad, half), jnp.float32),
            pltpu.SemaphoreType.DMA,
            pltpu.SemaphoreType.DMA,
            pltpu.SemaphoreType.DMA,
            pltpu.SemaphoreType.DMA,
        ],
    )
    gcn = gcn_call(xws.reshape(2 * n, half), idx_pack, zeros2)

    out = pl.pallas_call(
        _combine_body,
        grid=(nb,),
        in_specs=[
            pl.BlockSpec((r, d), lambda i: (i, 0)),
            pl.BlockSpec((r, d), lambda i: (i, 0)),
            pl.BlockSpec((2, r, half), lambda i: (0, i, 0)),
            pl.BlockSpec((r, 2), lambda i: (i, 0)),
            pl.BlockSpec((1, d), lambda i: (0, 0)),
        ],
        out_specs=pl.BlockSpec((r, d), lambda i: (i, 0)),
        out_shape=jax.ShapeDtypeStruct((n, d), jnp.float32),
    )(x, h2, gcn, degp_t, bias.reshape(1, d))
    return out


# speedup vs baseline: 15.0046x; 1.6371x over previous
import functools

import jax
import jax.numpy as jnp
from jax import lax
from jax.experimental import pallas as pl
from jax.experimental.pallas import tpu as pltpu
from jax.experimental.pallas import tpu_sc as plsc

GAMMA = 0.1
EPSILON = 0.1

NC = 2
NS = 16
K = 120
NI = 6
NG = 3
ZR = 80


@functools.cache
def _sc_mesh():
    return plsc.VectorSubcoreMesh(core_axis_name="core",
                                  subcore_axis_name="subcore",
                                  num_cores=NC, num_subcores=NS)


def _deg_body(npad, nch_deg, cols_hbm, ones_hbm, zeros_hbm, degp_hbm,
              cols_v, ones_v, zbuf, deg_sh):
    stripe = npad // NS
    c = lax.axis_index("core")
    s = lax.axis_index("subcore")
    pltpu.sync_copy(zeros_hbm, zbuf)
    pltpu.sync_copy(zbuf, deg_sh.at[pl.ds(s * stripe, stripe)])
    pltpu.sync_copy(cols_hbm.at[c, s], cols_v)
    pltpu.sync_copy(ones_hbm, ones_v)
    plsc.subcore_barrier()

    @pl.loop(0, nch_deg)
    def _(j):
        pltpu.sync_copy(ones_v, deg_sh.at[cols_v.at[j]], add=True)

    plsc.subcore_barrier()
    pltpu.sync_copy(deg_sh.at[pl.ds(s * stripe, stripe)], zbuf)
    pltpu.sync_copy(zbuf, degp_hbm.at[pl.ds(c * npad + s * stripe, stripe)])


def _gcn_body(npad, nch, zrows, xws_hbm, idx_hbm, zeros_hbm, gcn_hbm,
              ib, gb, acc_sh, isems, gsems, ssems):
    stripe = npad // NS
    ni = len(ib)
    ng = len(gb)
    c = lax.axis_index("core")
    s = lax.axis_index("subcore")
    pltpu.sync_copy(zeros_hbm, gb[0].at[pl.ds(0, zrows)])

    @pl.loop(0, stripe, step=zrows)
    def _(i):
        pltpu.sync_copy(gb[0].at[pl.ds(0, zrows)],
                        acc_sh.at[pl.ds(s * stripe + i, zrows)])

    plsc.subcore_barrier()

    for t in range(ni // 2):
        pltpu.async_copy(idx_hbm.at[c, s, t], ib[t], isems[t])

    @pl.loop(0, nch, step=ni)
    def _(j):
        for u in range(ni):
            t = j + u
            tg = (u + 1) % ng
            if u >= 3:
                pltpu.make_async_copy(gb[u % ng],
                                      acc_sh.at[ib[(u + 3) % ni].at[1]],
                                      ssems[u % ng]).wait()
            else:
                @pl.when(t >= 3)
                def _():
                    pltpu.make_async_copy(gb[u % ng],
                                          acc_sh.at[ib[(u + 3) % ni].at[1]],
                                          ssems[u % ng]).wait()
            nxt = t + ni // 2
            iu = (u + ni // 2) % ni

            @pl.when(nxt < nch)
            def _():
                pltpu.async_copy(idx_hbm.at[c, s, nxt], ib[iu], isems[iu])

            if u >= 2:
                pltpu.make_async_copy(xws_hbm.at[ib[(u - 2) % ni].at[0]],
                                      gb[tg], gsems[tg]).wait()
                pltpu.async_copy(gb[tg], acc_sh.at[ib[(u - 2) % ni].at[1]],
                                ssems[tg], add=True)
            else:
                @pl.when(t >= 2)
                def _():
                    pltpu.make_async_copy(xws_hbm.at[ib[(u - 2) % ni].at[0]],
                                          gb[tg], gsems[tg]).wait()
                    pltpu.async_copy(gb[tg],
                                     acc_sh.at[ib[(u - 2) % ni].at[1]],
                                     ssems[tg], add=True)

            pltpu.make_async_copy(idx_hbm.at[c, s, t], ib[u % ni],
                                  isems[u % ni]).wait()
            pltpu.async_copy(xws_hbm.at[ib[u % ni].at[0]], gb[u % ng],
                             gsems[u % ng])

    pltpu.make_async_copy(gb[(nch - 3) % ng],
                          acc_sh.at[ib[(nch - 3) % ni].at[1]],
                          ssems[(nch - 3) % ng]).wait()
    for t in (nch - 2, nch - 1):
        pltpu.make_async_copy(xws_hbm.at[ib[t % ni].at[0]], gb[t % ng],
                              gsems[t % ng]).wait()
        pltpu.sync_copy(gb[t % ng], acc_sh.at[ib[t % ni].at[1]], add=True)

    plsc.subcore_barrier()

    @pl.loop(0, stripe, step=2 * zrows)
    def _(i):
        pltpu.sync_copy(acc_sh.at[pl.ds(s * stripe + i, zrows)],
                        gb[0].at[pl.ds(0, zrows)])
        pltpu.sync_copy(gb[0].at[pl.ds(0, zrows)],
                        gcn_hbm.at[c, pl.ds(s * stripe + i, zrows)])
        pltpu.sync_copy(acc_sh.at[pl.ds(s * stripe + i + zrows, zrows)],
                        gb[1].at[pl.ds(0, zrows)])
        pltpu.sync_copy(gb[1].at[pl.ds(0, zrows)],
                        gcn_hbm.at[c, pl.ds(s * stripe + i + zrows, zrows)])


def _dense_body(x_ref, wcat_ref, degp_ref, h2_ref, xws_ref):
    xb = x_ref[...]
    m = jnp.dot(xb, wcat_ref[...], preferred_element_type=jnp.float32)
    d = xb.shape[1]
    h2_ref[...] = m[:, d:]
    deg = degp_ref[:, 0:1] + degp_ref[:, 1:2]
    dis = jnp.where(deg > 0.0, lax.rsqrt(deg), 0.0)
    xw = m[:, :d] * dis
    half = d // 2
    xws_ref[0] = xw[:, :half]
    xws_ref[1] = xw[:, half:]


def _combine_body(x_ref, h2_ref, gcn_ref, degp_ref, bias_ref, o_ref):
    deg = degp_ref[:, 0:1] + degp_ref[:, 1:2]
    dis = jnp.where(deg > 0.0, lax.rsqrt(deg), 0.0)
    g = jnp.concatenate([gcn_ref[0], gcn_ref[1]], axis=1)
    h = h2_ref[...] + g * dis + bias_ref[...]
    o_ref[...] = x_ref[...] + EPSILON * jnp.tanh(h)


def kernel(x, edge_index, W, W_phi, bias):
    n, d = x.shape
    e = edge_index.shape[1]
    half = d // 2
    npad = ((n + 2 * ZR * NS - 1) // (2 * ZR * NS)) * (2 * ZR * NS)
    stripe = npad // NS
    nch = NI * ((e + NI * NS * K - 1) // (NI * NS * K))
    epad = NS * K * nch
    nch_deg = epad // (NC * NS * K)

    ei = edge_index.astype(jnp.int32)
    rows = jnp.concatenate([ei[0], jnp.zeros((epad - e,), jnp.int32)])
    cols = jnp.concatenate([ei[1],
                            jnp.full((epad - e,), npad - 1, jnp.int32)])
    rows3 = rows.reshape(NS, nch, K)
    cols3 = cols.reshape(NS, nch, K)
    idx_pack = jnp.stack(
        [jnp.stack([rows3, cols3], axis=2),
         jnp.stack([rows3 + n, cols3], axis=2)], axis=0)
    cols_deg = cols.reshape(NC, NS, nch_deg, K)

    ones128 = jnp.ones((K,), jnp.float32)
    zeros1 = jnp.zeros((stripe,), jnp.float32)
    zeros2 = jnp.zeros((ZR, half), jnp.float32)

    wcat = jnp.concatenate(
        [W_phi.T, (W - W.T - GAMMA * jnp.eye(d, dtype=x.dtype)).T], axis=1)

    deg_call = pl.kernel(
        functools.partial(_deg_body, npad, nch_deg),
        out_type=jax.ShapeDtypeStruct((NC * npad,), jnp.float32),
        mesh=_sc_mesh(),
        scratch_types=[
            pltpu.VMEM((nch_deg, K), jnp.int32),
            pltpu.VMEM((K,), jnp.float32),
            pltpu.VMEM((stripe,), jnp.float32),
            pltpu.VMEM_SHARED((npad,), jnp.float32),
        ],
    )
    degp = deg_call(cols_deg, ones128, zeros1)
    degp_t = degp.reshape(NC, npad).T

    nb = 10
    r = n // nb
    h2, xws = pl.pallas_call(
        _dense_body,
        grid=(nb,),
        in_specs=[
            pl.BlockSpec((r, d), lambda i: (i, 0)),
            pl.BlockSpec((d, 2 * d), lambda i: (0, 0)),
            pl.BlockSpec((r, 2), lambda i: (i, 0)),
        ],
        out_specs=[
            pl.BlockSpec((r, d), lambda i: (i, 0)),
            pl.BlockSpec((2, r, half), lambda i: (0, i, 0)),
        ],
        out_shape=[
            jax.ShapeDtypeStruct((n, d), jnp.float32),
            jax.ShapeDtypeStruct((2, n, half), jnp.float32),
        ],
    )(x, wcat, degp_t)

    gcn_call = pl.kernel(
        functools.partial(_gcn_body, npad, nch, ZR),
        out_type=jax.ShapeDtypeStruct((NC, npad, half), jnp.float32),
        mesh=_sc_mesh(),
        scratch_types=[
            [pltpu.VMEM((2, K), jnp.int32) for _ in range(NI)],
            [pltpu.VMEM((K, half), jnp.float32) for _ in range(NG)],
            pltpu.VMEM_SHARED((npad, half), jnp.float32),
            [pltpu.SemaphoreType.DMA for _ in range(NI)],
            [pltpu.SemaphoreType.DMA for _ in range(NG)],
            [pltpu.SemaphoreType.DMA for _ in range(NG)],
        ],
    )
    gcn = gcn_call(xws.reshape(2 * n, half), idx_pack, zeros2)

    out = pl.pallas_call(
        _combine_body,
        grid=(nb,),
        in_specs=[
            pl.BlockSpec((r, d), lambda i: (i, 0)),
            pl.BlockSpec((r, d), lambda i: (i, 0)),
            pl.BlockSpec((2, r, half), lambda i: (0, i, 0)),
            pl.BlockSpec((r, 2), lambda i: (i, 0)),
            pl.BlockSpec((1, d), lambda i: (0, 0)),
        ],
        out_specs=pl.BlockSpec((r, d), lambda i: (i, 0)),
        out_shape=jax.ShapeDtypeStruct((n, d), jnp.float32),
    )(x, h2, gcn, degp_t, bias.reshape(1, d))
    return out
